# Initial kernel scaffold; baseline (speedup 1.0000x reference)
import jax
import jax.numpy as jnp
from jax.experimental import pallas as pl


def kernel(x, adj, W1, a1_src, a1_dst, W2, a2_src, a2_dst, Wd, bd):
    raise NotImplementedError("write your pallas kernel here")


# speedup vs baseline: 2.1512x; 2.1512x over previous
import jax
import jax.numpy as jnp
from jax.experimental import pallas as pl
from jax.experimental.pallas import tpu as pltpu

_N = 2048
_H = 6
_F1 = 16
_F2 = 24
_D1 = _H * _F1
_D2 = _H * _F2
_R = 256
_NB = _N // _R
_NEG = -1e9


def _lrelu(v):
    return jnp.where(v > 0, v, 0.2 * v)


def _elu(v):
    return jnp.where(v > 0, v, jnp.exp(v) - 1.0)


def _fused(x_ref, adj_ref, w1_ref, a1s_ref, a1d_ref, w2_ref, a2s_ref,
           a2d_ref, wdt_ref, bd_ref, out_ref,
           hp1, es1, ed1, h1s, hp2, es2, ed2, pooled):
    step = pl.program_id(0)
    thresh = 32.0 / _N

    @pl.when(step == 0)
    def _proj1():
        xv = x_ref[...]
        for h in range(_H):
            hp = jnp.dot(xv, w1_ref[h], preferred_element_type=jnp.float32)
            hp1[h] = hp
            es1[:, h:h + 1] = jax.lax.dot_general(
                hp, a1s_ref[h:h + 1, :], (((1,), (1,)), ((), ())),
                preferred_element_type=jnp.float32)
            ed1[h:h + 1, :] = jax.lax.dot_general(
                a1d_ref[h:h + 1, :], hp, (((1,), (1,)), ((), ())),
                preferred_element_type=jnp.float32)
        pooled[...] = jnp.zeros_like(pooled)

    @pl.when(step < _NB)
    def _layer1():
        row0 = step * _R
        adjb = adj_ref[...]
        col = jax.lax.broadcasted_iota(jnp.int32, (_R, _N), 1)
        row = row0 + jax.lax.broadcasted_iota(jnp.int32, (_R, _N), 0)
        m = jnp.logical_or(adjb < thresh, col == row)
        for h in range(_H):
            e = es1[pl.ds(row0, _R), h:h + 1] + ed1[h:h + 1, :]
            e = jnp.where(m, _lrelu(e), _NEG)
            mx = jnp.max(e, axis=1, keepdims=True)
            p = jnp.exp(e - mx)
            s = jnp.sum(p, axis=1, keepdims=True)
            o = jnp.dot(p, hp1[h], preferred_element_type=jnp.float32)
            h1s[pl.ds(row0, _R), h * _F1:(h + 1) * _F1] = _elu(o / s)

    @pl.when(step == _NB)
    def _proj2():
        h1 = h1s[...]
        for h in range(_H):
            hp = jnp.dot(h1, w2_ref[h], preferred_element_type=jnp.float32)
            hp2[h] = hp
            es2[:, h:h + 1] = jax.lax.dot_general(
                hp, a2s_ref[h:h + 1, :], (((1,), (1,)), ((), ())),
                preferred_element_type=jnp.float32)
            ed2[h:h + 1, :] = jax.lax.dot_general(
                a2d_ref[h:h + 1, :], hp, (((1,), (1,)), ((), ())),
                preferred_element_type=jnp.float32)

    @pl.when(step >= _NB)
    def _layer2():
        row0 = (step - _NB) * _R
        adjb = adj_ref[...]
        col = jax.lax.broadcasted_iota(jnp.int32, (_R, _N), 1)
        row = row0 + jax.lax.broadcasted_iota(jnp.int32, (_R, _N), 0)
        m = jnp.logical_or(adjb < thresh, col == row)
        for h in range(_H):
            e = es2[pl.ds(row0, _R), h:h + 1] + ed2[h:h + 1, :]
            e = jnp.where(m, _lrelu(e), _NEG)
            mx = jnp.max(e, axis=1, keepdims=True)
            p = jnp.exp(e - mx)
            s = jnp.sum(p, axis=1, keepdims=True)
            o = jnp.dot(p, hp2[h], preferred_element_type=jnp.float32)
            o = _elu(o / s)
            pooled[0:1, h * _F2:(h + 1) * _F2] += jnp.sum(
                o, axis=0, keepdims=True)

    @pl.when(step == 2 * _NB - 1)
    def _final():
        pv = pooled[0:1, :]
        denom = jnp.maximum(
            jnp.sqrt(jnp.sum(pv * pv, axis=1, keepdims=True)), 1e-12)
        out_ref[...] = (jnp.sum((pv / denom) * wdt_ref[...], axis=1,
                                keepdims=True) + bd_ref[...])


def _build():
    return pl.pallas_call(
        _fused,
        grid=(2 * _NB,),
        in_specs=[
            pl.BlockSpec((_N, 11), lambda i: (0, 0)),
            pl.BlockSpec((_R, _N), lambda i: (i % _NB, 0)),
            pl.BlockSpec((_H, 11, _F1), lambda i: (0, 0, 0)),
            pl.BlockSpec((_H, _F1), lambda i: (0, 0)),
            pl.BlockSpec((_H, _F1), lambda i: (0, 0)),
            pl.BlockSpec((_H, _D1, _F2), lambda i: (0, 0, 0)),
            pl.BlockSpec((_H, _F2), lambda i: (0, 0)),
            pl.BlockSpec((_H, _F2), lambda i: (0, 0)),
            pl.BlockSpec((1, _D2), lambda i: (0, 0)),
            pl.BlockSpec((1, 1), lambda i: (0, 0)),
        ],
        out_specs=pl.BlockSpec((1, 1), lambda i: (0, 0)),
        out_shape=jax.ShapeDtypeStruct((1, 1), jnp.float32),
        scratch_shapes=[
            pltpu.VMEM((_H, _N, _F1), jnp.float32),
            pltpu.VMEM((_N, _H), jnp.float32),
            pltpu.VMEM((_H, _N), jnp.float32),
            pltpu.VMEM((_N, _D1), jnp.float32),
            pltpu.VMEM((_H, _N, _F2), jnp.float32),
            pltpu.VMEM((_N, _H), jnp.float32),
            pltpu.VMEM((_H, _N), jnp.float32),
            pltpu.VMEM((8, _D2), jnp.float32),
        ],
    )


def kernel(x, adj, W1, a1_src, a1_dst, W2, a2_src, a2_dst, Wd, bd):
    out = _build()(x, adj, W1, a1_src, a1_dst, W2, a2_src, a2_dst,
                   Wd.reshape(1, _D2), bd.reshape(1, 1))
    return out.reshape(1)


# speedup vs baseline: 3.0093x; 1.3989x over previous
import jax
import jax.numpy as jnp
from jax.experimental import pallas as pl
from jax.experimental.pallas import tpu as pltpu

_N = 2048
_H = 6
_F1 = 16
_F2 = 24
_D1 = _H * _F1
_D2 = _H * _F2
_R = 256
_NB = _N // _R
_NEG = -1e9


def _lrelu(v):
    return jnp.maximum(v, 0.2 * v)


def _elu(v):
    return jnp.where(v > 0, v, jnp.exp(v) - 1.0)


def _fused(x_ref, adj_ref, w1_ref, a1s_ref, a1d_ref, w2_ref, a2s_ref,
           a2d_ref, wdt_ref, bd_ref, out_ref,
           hp1, es1, ed1, bs1, h1s, hp2, es2, ed2, bs2, pooled):
    step = pl.program_id(0)
    thresh = 32.0 / _N

    @pl.when(step == 0)
    def _proj1():
        xv = x_ref[...]
        for h in range(_H):
            hp = jnp.dot(xv, w1_ref[h], preferred_element_type=jnp.float32)
            hp1[h, :, 0:_F1] = hp
            hp1[h, :, _F1:_F1 + 1] = jnp.ones((_N, 1), jnp.float32)
            esh = jax.lax.dot_general(
                hp, a1s_ref[h:h + 1, :], (((1,), (1,)), ((), ())),
                preferred_element_type=jnp.float32)
            es1[:, h:h + 1] = esh
            edh = jax.lax.dot_general(
                a1d_ref[h:h + 1, :], hp, (((1,), (1,)), ((), ())),
                preferred_element_type=jnp.float32)
            ed1[h:h + 1, :] = edh
            bs1[:, h:h + 1] = _lrelu(esh + jnp.max(edh))
        pooled[...] = jnp.zeros_like(pooled)

    @pl.when(step < _NB)
    def _layer1():
        row0 = step * _R
        adjb = adj_ref[...]
        col = jax.lax.broadcasted_iota(jnp.int32, (_R, _N), 1)
        row = row0 + jax.lax.broadcasted_iota(jnp.int32, (_R, _N), 0)
        m = jnp.logical_or(adjb < thresh, col == row)
        for h in range(_H):
            e = es1[pl.ds(row0, _R), h:h + 1] + ed1[h:h + 1, :]
            e = jnp.where(m, _lrelu(e) - bs1[pl.ds(row0, _R), h:h + 1], _NEG)
            p = jnp.exp(e)
            o = jnp.dot(p, hp1[h], preferred_element_type=jnp.float32)
            h1s[pl.ds(row0, _R), h * _F1:(h + 1) * _F1] = _elu(
                o[:, 0:_F1] / o[:, _F1:_F1 + 1])

    @pl.when(step == _NB)
    def _proj2():
        h1 = h1s[...]
        for h in range(_H):
            hp = jnp.dot(h1, w2_ref[h], preferred_element_type=jnp.float32)
            hp2[h, :, 0:_F2] = hp
            hp2[h, :, _F2:_F2 + 1] = jnp.ones((_N, 1), jnp.float32)
            esh = jax.lax.dot_general(
                hp, a2s_ref[h:h + 1, :], (((1,), (1,)), ((), ())),
                preferred_element_type=jnp.float32)
            es2[:, h:h + 1] = esh
            edh = jax.lax.dot_general(
                a2d_ref[h:h + 1, :], hp, (((1,), (1,)), ((), ())),
                preferred_element_type=jnp.float32)
            ed2[h:h + 1, :] = edh
            bs2[:, h:h + 1] = _lrelu(esh + jnp.max(edh))

    @pl.when(step >= _NB)
    def _layer2():
        row0 = (step - _NB) * _R
        adjb = adj_ref[...]
        col = jax.lax.broadcasted_iota(jnp.int32, (_R, _N), 1)
        row = row0 + jax.lax.broadcasted_iota(jnp.int32, (_R, _N), 0)
        m = jnp.logical_or(adjb < thresh, col == row)
        for h in range(_H):
            e = es2[pl.ds(row0, _R), h:h + 1] + ed2[h:h + 1, :]
            e = jnp.where(m, _lrelu(e) - bs2[pl.ds(row0, _R), h:h + 1], _NEG)
            p = jnp.exp(e)
            o = jnp.dot(p, hp2[h], preferred_element_type=jnp.float32)
            o = _elu(o[:, 0:_F2] / o[:, _F2:_F2 + 1])
            pooled[0:1, h * _F2:(h + 1) * _F2] += jnp.sum(
                o, axis=0, keepdims=True)

    @pl.when(step == 2 * _NB - 1)
    def _final():
        pv = pooled[0:1, :]
        denom = jnp.maximum(
            jnp.sqrt(jnp.sum(pv * pv, axis=1, keepdims=True)), 1e-12)
        out_ref[...] = (jnp.sum((pv / denom) * wdt_ref[...], axis=1,
                                keepdims=True) + bd_ref[...])


def _build():
    return pl.pallas_call(
        _fused,
        grid=(2 * _NB,),
        in_specs=[
            pl.BlockSpec((_N, 11), lambda i: (0, 0)),
            pl.BlockSpec((_R, _N), lambda i: (i % _NB, 0)),
            pl.BlockSpec((_H, 11, _F1), lambda i: (0, 0, 0)),
            pl.BlockSpec((_H, _F1), lambda i: (0, 0)),
            pl.BlockSpec((_H, _F1), lambda i: (0, 0)),
            pl.BlockSpec((_H, _D1, _F2), lambda i: (0, 0, 0)),
            pl.BlockSpec((_H, _F2), lambda i: (0, 0)),
            pl.BlockSpec((_H, _F2), lambda i: (0, 0)),
            pl.BlockSpec((1, _D2), lambda i: (0, 0)),
            pl.BlockSpec((1, 1), lambda i: (0, 0)),
        ],
        out_specs=pl.BlockSpec((1, 1), lambda i: (0, 0)),
        out_shape=jax.ShapeDtypeStruct((1, 1), jnp.float32),
        scratch_shapes=[
            pltpu.VMEM((_H, _N, _F1 + 1), jnp.float32),
            pltpu.VMEM((_N, _H), jnp.float32),
            pltpu.VMEM((_H, _N), jnp.float32),
            pltpu.VMEM((_N, _H), jnp.float32),
            pltpu.VMEM((_N, _D1), jnp.float32),
            pltpu.VMEM((_H, _N, _F2 + 1), jnp.float32),
            pltpu.VMEM((_N, _H), jnp.float32),
            pltpu.VMEM((_H, _N), jnp.float32),
            pltpu.VMEM((_N, _H), jnp.float32),
            pltpu.VMEM((8, _D2), jnp.float32),
        ],
    )


def kernel(x, adj, W1, a1_src, a1_dst, W2, a2_src, a2_dst, Wd, bd):
    out = _build()(x, adj, W1, a1_src, a1_dst, W2, a2_src, a2_dst,
                   Wd.reshape(1, _D2), bd.reshape(1, 1))
    return out.reshape(1)


# speedup vs baseline: 3.4363x; 1.1419x over previous
import jax
import jax.numpy as jnp
from jax.experimental import pallas as pl
from jax.experimental.pallas import tpu as pltpu

_N = 2048
_H = 6
_F1 = 16
_F2 = 24
_D1 = _H * _F1
_D2 = _H * _F2
_R = 256
_NB = _N // _R
_NEG = -1e9


def _lrelu(v):
    return jnp.maximum(v, 0.2 * v)


def _elu(v):
    return jnp.where(v > 0, v, jnp.exp(v) - 1.0)


def _fused(x_ref, adj_ref, w1_ref, a1s_ref, a1d_ref, w2_ref, a2s_ref,
           a2d_ref, wdt_ref, bd_ref, out_ref,
           hp1, es1, ed1, bs1, h1s, hp2, es2, ed2, bs2, pooled):
    step = pl.program_id(0)
    thresh = 32.0 / _N

    @pl.when(step == 0)
    def _proj1():
        xv = x_ref[...]
        for h in range(_H):
            hp = jnp.dot(xv, w1_ref[h], preferred_element_type=jnp.float32)
            hp1[h, :, 0:_F1] = hp.astype(jnp.bfloat16)
            hp1[h, :, _F1:_F1 + 1] = jnp.ones((_N, 1), jnp.bfloat16)
            esh = jax.lax.dot_general(
                hp, a1s_ref[h:h + 1, :], (((1,), (1,)), ((), ())),
                preferred_element_type=jnp.float32)
            es1[:, h:h + 1] = esh.astype(jnp.bfloat16)
            edh = jax.lax.dot_general(
                a1d_ref[h:h + 1, :], hp, (((1,), (1,)), ((), ())),
                preferred_element_type=jnp.float32)
            ed1[h:h + 1, :] = edh.astype(jnp.bfloat16)
            bs1[:, h:h + 1] = _lrelu(esh + jnp.max(edh)).astype(jnp.bfloat16)
        pooled[...] = jnp.zeros_like(pooled)

    @pl.when(step < _NB)
    def _layer1():
        row0 = step * _R
        adjb = adj_ref[...]
        col = jax.lax.broadcasted_iota(jnp.int32, (_R, _N), 1)
        row = row0 + jax.lax.broadcasted_iota(jnp.int32, (_R, _N), 0)
        m = jnp.logical_or(adjb < thresh, col == row)
        for h in range(_H):
            e = es1[pl.ds(row0, _R), h:h + 1] + ed1[h:h + 1, :]
            e = jnp.where(m, _lrelu(e) - bs1[pl.ds(row0, _R), h:h + 1], _NEG)
            p = jnp.exp(e)
            o = jnp.dot(p, hp1[h], preferred_element_type=jnp.float32)
            h1s[pl.ds(row0, _R), h * _F1:(h + 1) * _F1] = _elu(
                o[:, 0:_F1] / o[:, _F1:_F1 + 1])

    @pl.when(step == _NB)
    def _proj2():
        h1 = h1s[...]
        for h in range(_H):
            hp = jnp.dot(h1, w2_ref[h], preferred_element_type=jnp.float32)
            hp2[h, :, 0:_F2] = hp.astype(jnp.bfloat16)
            hp2[h, :, _F2:_F2 + 1] = jnp.ones((_N, 1), jnp.bfloat16)
            esh = jax.lax.dot_general(
                hp, a2s_ref[h:h + 1, :], (((1,), (1,)), ((), ())),
                preferred_element_type=jnp.float32)
            es2[:, h:h + 1] = esh.astype(jnp.bfloat16)
            edh = jax.lax.dot_general(
                a2d_ref[h:h + 1, :], hp, (((1,), (1,)), ((), ())),
                preferred_element_type=jnp.float32)
            ed2[h:h + 1, :] = edh.astype(jnp.bfloat16)
            bs2[:, h:h + 1] = _lrelu(esh + jnp.max(edh)).astype(jnp.bfloat16)

    @pl.when(step >= _NB)
    def _layer2():
        row0 = (step - _NB) * _R
        adjb = adj_ref[...]
        col = jax.lax.broadcasted_iota(jnp.int32, (_R, _N), 1)
        row = row0 + jax.lax.broadcasted_iota(jnp.int32, (_R, _N), 0)
        m = jnp.logical_or(adjb < thresh, col == row)
        for h in range(_H):
            e = es2[pl.ds(row0, _R), h:h + 1] + ed2[h:h + 1, :]
            e = jnp.where(m, _lrelu(e) - bs2[pl.ds(row0, _R), h:h + 1], _NEG)
            p = jnp.exp(e)
            o = jnp.dot(p, hp2[h], preferred_element_type=jnp.float32)
            o = _elu(o[:, 0:_F2] / o[:, _F2:_F2 + 1])
            pooled[0:1, h * _F2:(h + 1) * _F2] += jnp.sum(
                o, axis=0, keepdims=True)

    @pl.when(step == 2 * _NB - 1)
    def _final():
        pv = pooled[0:1, :]
        denom = jnp.maximum(
            jnp.sqrt(jnp.sum(pv * pv, axis=1, keepdims=True)), 1e-12)
        out_ref[...] = (jnp.sum((pv / denom) * wdt_ref[...], axis=1,
                                keepdims=True) + bd_ref[...])


def _build():
    return pl.pallas_call(
        _fused,
        grid=(2 * _NB,),
        in_specs=[
            pl.BlockSpec((_N, 11), lambda i: (0, 0)),
            pl.BlockSpec((_R, _N), lambda i: (i % _NB, 0)),
            pl.BlockSpec((_H, 11, _F1), lambda i: (0, 0, 0)),
            pl.BlockSpec((_H, _F1), lambda i: (0, 0)),
            pl.BlockSpec((_H, _F1), lambda i: (0, 0)),
            pl.BlockSpec((_H, _D1, _F2), lambda i: (0, 0, 0)),
            pl.BlockSpec((_H, _F2), lambda i: (0, 0)),
            pl.BlockSpec((_H, _F2), lambda i: (0, 0)),
            pl.BlockSpec((1, _D2), lambda i: (0, 0)),
            pl.BlockSpec((1, 1), lambda i: (0, 0)),
        ],
        out_specs=pl.BlockSpec((1, 1), lambda i: (0, 0)),
        out_shape=jax.ShapeDtypeStruct((1, 1), jnp.float32),
        scratch_shapes=[
            pltpu.VMEM((_H, _N, _F1 + 1), jnp.bfloat16),
            pltpu.VMEM((_N, _H), jnp.bfloat16),
            pltpu.VMEM((_H, _N), jnp.bfloat16),
            pltpu.VMEM((_N, _H), jnp.bfloat16),
            pltpu.VMEM((_N, _D1), jnp.float32),
            pltpu.VMEM((_H, _N, _F2 + 1), jnp.bfloat16),
            pltpu.VMEM((_N, _H), jnp.bfloat16),
            pltpu.VMEM((_H, _N), jnp.bfloat16),
            pltpu.VMEM((_N, _H), jnp.bfloat16),
            pltpu.VMEM((8, _D2), jnp.float32),
        ],
    )


def kernel(x, adj, W1, a1_src, a1_dst, W2, a2_src, a2_dst, Wd, bd):
    out = _build()(x, adj, W1, a1_src, a1_dst, W2, a2_src, a2_dst,
                   Wd.reshape(1, _D2), bd.reshape(1, 1))
    return out.reshape(1)
